# baseline (speedup 1.0000x reference)
import functools

import jax
import jax.numpy as jnp
from jax import lax
from jax.experimental import pallas as pl
from jax.experimental.pallas import tpu as pltpu
from jax.experimental.pallas import tpu_sc as plsc

_BATCH = 1024
_HIST = 20
_D = 64
_N = _BATCH * _HIST
_V = 1000000
_NW = 32
_LANE = 128
_CPC = 4
_CHW = _CPC * _LANE
_NRING = 2
_MAXL = 64


def _make_sweep(V=_V, N=_N):
    mesh = plsc.VectorSubcoreMesh(core_axis_name="c", subcore_axis_name="s")
    nch = (V + _CHW - 1) // _CHW
    tcols = (V + _LANE - 1) // _LANE
    nout = N + 8
    dump = N
    nsub = N // 2560

    @functools.partial(
        pl.kernel,
        mesh=mesh,
        out_type=jax.ShapeDtypeStruct((nout, _LANE), jnp.float32),
        compiler_params=pltpu.CompilerParams(
            use_tc_tiling_on_sc=True, needs_layout_passes=False
        ),
        scratch_types=[
            pltpu.VMEM((2560,), jnp.int32),
            pltpu.VMEM((N + 16,), jnp.int32),
            pltpu.VMEM((N + 16,), jnp.int32),
            pltpu.VMEM((16 * _MAXL,), jnp.int32),
            pltpu.VMEM((_MAXL + 16,), jnp.int32),
            pltpu.VMEM((_NRING, _D, _CHW), jnp.float32),
            pltpu.VMEM((128, _LANE), jnp.float32),
            pltpu.VMEM((128,), jnp.int32),
            pltpu.VMEM((32,), jnp.int32),
            pltpu.VMEM((32,), jnp.int32),
            pltpu.SMEM((_MAXL,), jnp.int32),
            pltpu.SemaphoreType.DMA((_NRING,)),
            pltpu.SemaphoreType.DMA,
        ],
    )
    def k(wt, idx, out, idx_st, gv, gj, hist, incl, slab, rows, jb, stv, stj,
          scur, sems, ssem):
        t = lax.axis_index("s") * 2 + lax.axis_index("c")
        iota = lax.iota(jnp.int32, 16)
        lane0 = iota == 0
        zeros16 = iota & 0
        nl = jnp.where(t < nch - (nch // _NW) * _NW, nch // _NW + 1,
                       nch // _NW)

        for z in range(16 * _MAXL // 256):
            for q in range(16):
                hist[pl.ds(z * 256 + q * 16, 16)] = zeros16
        ones = zeros16 + 1
        for sub in range(nsub):
            pltpu.sync_copy(idx.at[pl.ds(sub * 2560, 2560)], idx_st)

            def hbody(r, c):
                vv = idx_st[pl.ds(r * 16, 16)]
                m = ((vv >> 9) & (_NW - 1)) == t
                lv = vv >> 14
                plsc.addupdate_scatter(hist, [iota * _MAXL + lv], ones, mask=m)
                return c

            lax.fori_loop(0, 160, hbody, jnp.int32(0))

        carry = jnp.int32(0)
        for q in range(_MAXL // 16):
            acc = zeros16
            for kk in range(16):
                acc = acc + hist[pl.ds(kk * _MAXL + q * 16, 16)]
            c = plsc.cumsum(acc) + carry
            incl[pl.ds(q * 16, 16)] = c
            carry = c[15]
        incl[pl.ds(_MAXL, 16)] = zeros16
        def cbody(l, c):
            prev = incl[pl.ds(jnp.maximum(l - 1, 0), 16)][0]
            scur[l] = jnp.where(l == 0, 0, prev)
            return c

        lax.fori_loop(0, _MAXL, cbody, jnp.int32(0))

        for sub in range(nsub):
            pltpu.sync_copy(idx.at[pl.ds(sub * 2560, 2560)], idx_st)

            def pbody(r, c, sub=sub):
                vv = idx_st[pl.ds(r * 16, 16)]
                jj = iota + (sub * 2560) + r * 16
                m = ((vv >> 9) & (_NW - 1)) == t
                cnt = plsc.all_reduce_population_count(m)[0]
                plsc.store_compressed(stv.at[pl.ds(0, 16)], vv, mask=m)
                plsc.store_compressed(stj.at[pl.ds(0, 16)], jj, mask=m)

                def place(i, c):
                    v0 = stv[pl.ds(i, 16)][0]
                    j0 = stj[pl.ds(i, 16)][0]
                    l0 = v0 >> 14
                    p = scur[l0]
                    scur[l0] = p + 1
                    plsc.store_scatter(gv, [zeros16 + p], zeros16 + v0,
                                       mask=lane0)
                    plsc.store_scatter(gj, [zeros16 + p], zeros16 + j0,
                                       mask=lane0)
                    return c

                return lax.fori_loop(0, cnt, place, c)

            lax.fori_loop(0, 160, pbody, jnp.int32(0))

        for g8 in range(8):
            jb[pl.ds(g8 * 16, 16)] = zeros16 + dump

        e_vec = [iota + 16 * gi for gi in range(4)]
        partial_tail = (V % _CHW) != 0

        def fetch(g, buf):
            col0 = pl.multiple_of(g * _CHW, _LANE)
            if partial_tail:
                @pl.when(g < nch - 1)
                def _():
                    pltpu.async_copy(
                        wt.at[:, pl.ds(col0, _CHW)], slab.at[buf], sems.at[buf]
                    )

                @pl.when(g == nch - 1)
                def _():
                    pltpu.async_copy(
                        wt.at[:, pl.ds(col0, _LANE)],
                        slab.at[buf, :, pl.ds(0, _LANE)], sems.at[buf]
                    )
            else:
                pltpu.async_copy(
                    wt.at[:, pl.ds(col0, _CHW)], slab.at[buf], sems.at[buf]
                )

        def wait_slab(g, buf):
            if partial_tail:
                @pl.when(g < nch - 1)
                def _():
                    pltpu.make_async_copy(
                        wt.at[:, pl.ds(0, _CHW)], slab.at[buf], sems.at[buf]
                    ).wait()

                @pl.when(g == nch - 1)
                def _():
                    pltpu.make_async_copy(
                        wt.at[:, pl.ds(0, _LANE)],
                        slab.at[buf, :, pl.ds(0, _LANE)], sems.at[buf]
                    ).wait()
            else:
                pltpu.make_async_copy(
                    wt.at[:, pl.ds(0, _CHW)], slab.at[buf], sems.at[buf]
                ).wait()

        def flush():
            pltpu.async_copy(rows, out.at[jb], ssem).wait()
            for g8 in range(8):
                jb[pl.ds(g8 * 16, 16)] = zeros16 + dump

        for p in range(_NRING - 1):
            fetch(p * _NW + t, p)

        def chunk_body(l, slot):
            buf = l & (_NRING - 1)
            g = l * _NW + t

            @pl.when(l + (_NRING - 1) < nl)
            def _():
                fetch((l + (_NRING - 1)) * _NW + t,
                      (l + (_NRING - 1)) & (_NRING - 1))

            wait_slab(g, buf)
            lo = jnp.where(l == 0, 0,
                           incl[pl.ds(jnp.maximum(l - 1, 0), 16)][0])
            hi = incl[pl.ds(l, 16)][0]
            bufv = zeros16 + buf
            base = g * _CHW

            def ebody(i, slot):
                mv0 = gv[pl.ds(i, 16)][0]
                mjv = gj[pl.ds(i, 16)]
                wv = zeros16 + (mv0 - base)
                for gi in range(4):
                    col = plsc.load_gather(slab, [bufv, e_vec[gi], wv])
                    rows.at[slot][pl.ds(gi * 16, 16)] = col
                plsc.store_scatter(jb, [zeros16 + slot], mjv, mask=lane0)
                slot = slot + 1

                @pl.when(slot == 128)
                def _():
                    flush()

                return jnp.where(slot == 128, 0, slot)

            return lax.fori_loop(lo, hi, ebody, slot)

        lax.fori_loop(0, nl, chunk_body, jnp.int32(0))
        flush()

    return k


_sweep = _make_sweep()


def kernel(input, weight):
    wt = weight.T
    idx = input.reshape(_N).astype(jnp.int32)
    out = _sweep(wt, idx)
    return out[:_N, :_D].reshape(_BATCH, _HIST, _D)


# speedup vs baseline: 1.1824x; 1.1824x over previous
import functools

import jax
import jax.numpy as jnp
from jax import lax
from jax.experimental import pallas as pl
from jax.experimental.pallas import tpu as pltpu
from jax.experimental.pallas import tpu_sc as plsc

_BATCH = 1024
_HIST = 20
_D = 64
_N = _BATCH * _HIST
_V = 1000000
_NW = 32
_LANE = 128
_CPC = 4
_CHW = _CPC * _LANE
_NRING = 3
_MAXL = 64


def _make_sweep(V=_V, N=_N):
    mesh = plsc.VectorSubcoreMesh(core_axis_name="c", subcore_axis_name="s")
    nch = (V + _CHW - 1) // _CHW
    tcols = (V + _LANE - 1) // _LANE
    nout = N + 8
    dump = N
    nsub = N // 2560

    @functools.partial(
        pl.kernel,
        mesh=mesh,
        out_type=jax.ShapeDtypeStruct((nout, _LANE), jnp.float32),
        compiler_params=pltpu.CompilerParams(
            use_tc_tiling_on_sc=True, needs_layout_passes=False
        ),
        scratch_types=[
            pltpu.VMEM((1280,), jnp.int32),
            pltpu.VMEM((N + 16,), jnp.int32),
            pltpu.VMEM((16 * _MAXL,), jnp.int32),
            pltpu.VMEM((_MAXL + 16,), jnp.int32),
            pltpu.VMEM((_NRING, _D, _CHW), jnp.float32),
            pltpu.VMEM((64, _LANE), jnp.float32),
            pltpu.VMEM((64,), jnp.int32),
            pltpu.VMEM((32,), jnp.int32),
            pltpu.VMEM((32,), jnp.int32),
            pltpu.SMEM((_MAXL,), jnp.int32),
            pltpu.SemaphoreType.DMA((_NRING,)),
            pltpu.SemaphoreType.DMA,
        ],
    )
    def k(wt, idx, out, idx_st, gp, hist, incl, slab, rows, jb, stv, stj,
          scur, sems, ssem):
        t = lax.axis_index("s") * 2 + lax.axis_index("c")
        iota = lax.iota(jnp.int32, 16)
        lane0 = iota == 0
        zeros16 = iota & 0
        nl = jnp.where(t < nch - (nch // _NW) * _NW, nch // _NW + 1,
                       nch // _NW)

        for z in range(16 * _MAXL // 256):
            for q in range(16):
                hist[pl.ds(z * 256 + q * 16, 16)] = zeros16
        ones = zeros16 + 1
        for sub in range(2 * nsub):
            pltpu.sync_copy(idx.at[pl.ds(sub * 1280, 1280)], idx_st)

            def hbody(r, c):
                vv = idx_st[pl.ds(r * 16, 16)]
                m = ((vv >> 9) & (_NW - 1)) == t
                lv = vv >> 14
                plsc.addupdate_scatter(hist, [iota * _MAXL + lv], ones, mask=m)
                return c

            lax.fori_loop(0, 80, hbody, jnp.int32(0))

        carry = jnp.int32(0)
        for q in range(_MAXL // 16):
            acc = zeros16
            for kk in range(16):
                acc = acc + hist[pl.ds(kk * _MAXL + q * 16, 16)]
            c = plsc.cumsum(acc) + carry
            incl[pl.ds(q * 16, 16)] = c
            carry = c[15]
        incl[pl.ds(_MAXL, 16)] = zeros16
        def cbody(l, c):
            prev = incl[pl.ds(jnp.maximum(l - 1, 0), 16)][0]
            scur[l] = jnp.where(l == 0, 0, prev)
            return c

        lax.fori_loop(0, _MAXL, cbody, jnp.int32(0))

        for sub in range(2 * nsub):
            pltpu.sync_copy(idx.at[pl.ds(sub * 1280, 1280)], idx_st)

            def pbody(r, c, sub=sub):
                vv = idx_st[pl.ds(r * 16, 16)]
                jj = iota + (sub * 1280) + r * 16
                m = ((vv >> 9) & (_NW - 1)) == t
                cnt = plsc.all_reduce_population_count(m)[0]
                pk = (jj << 9) | (vv & (_CHW - 1))
                plsc.store_compressed(stv.at[pl.ds(0, 16)], vv, mask=m)
                plsc.store_compressed(stj.at[pl.ds(0, 16)], pk, mask=m)

                def place(i, c):
                    v0 = stv[pl.ds(i, 16)][0]
                    p0 = stj[pl.ds(i, 16)][0]
                    l0 = v0 >> 14
                    p = scur[l0]
                    scur[l0] = p + 1
                    plsc.store_scatter(gp, [zeros16 + p], zeros16 + p0,
                                       mask=lane0)
                    return c

                return lax.fori_loop(0, cnt, place, c)

            lax.fori_loop(0, 80, pbody, jnp.int32(0))

        for g8 in range(4):
            jb[pl.ds(g8 * 16, 16)] = zeros16 + dump

        e_vec = [iota + 16 * gi for gi in range(4)]
        partial_tail = (V % _CHW) != 0

        def fetch(g, buf):
            col0 = pl.multiple_of(g * _CHW, _LANE)
            if partial_tail:
                @pl.when(g < nch - 1)
                def _():
                    pltpu.async_copy(
                        wt.at[:, pl.ds(col0, _CHW)], slab.at[buf], sems.at[buf]
                    )

                @pl.when(g == nch - 1)
                def _():
                    pltpu.async_copy(
                        wt.at[:, pl.ds(col0, _LANE)],
                        slab.at[buf, :, pl.ds(0, _LANE)], sems.at[buf]
                    )
            else:
                pltpu.async_copy(
                    wt.at[:, pl.ds(col0, _CHW)], slab.at[buf], sems.at[buf]
                )

        def wait_slab(g, buf):
            if partial_tail:
                @pl.when(g < nch - 1)
                def _():
                    pltpu.make_async_copy(
                        wt.at[:, pl.ds(0, _CHW)], slab.at[buf], sems.at[buf]
                    ).wait()

                @pl.when(g == nch - 1)
                def _():
                    pltpu.make_async_copy(
                        wt.at[:, pl.ds(0, _LANE)],
                        slab.at[buf, :, pl.ds(0, _LANE)], sems.at[buf]
                    ).wait()
            else:
                pltpu.make_async_copy(
                    wt.at[:, pl.ds(0, _CHW)], slab.at[buf], sems.at[buf]
                ).wait()

        def flush():
            pltpu.async_copy(rows, out.at[jb], ssem).wait()
            for g8 in range(4):
                jb[pl.ds(g8 * 16, 16)] = zeros16 + dump

        for p in range(_NRING - 1):
            fetch(p * _NW + t, p)

        def chunk_body(l, slot):
            buf = lax.rem(l, _NRING)
            g = l * _NW + t

            @pl.when(l + (_NRING - 1) < nl)
            def _():
                fetch((l + (_NRING - 1)) * _NW + t,
                      lax.rem(l + (_NRING - 1), _NRING))

            wait_slab(g, buf)
            lo = jnp.where(l == 0, 0,
                           incl[pl.ds(jnp.maximum(l - 1, 0), 16)][0])
            hi = incl[pl.ds(l, 16)][0]
            bufv = zeros16 + buf

            def ebody(i, slot):
                pk = gp[pl.ds(i, 16)][0]
                wv = zeros16 + (pk & (_CHW - 1))
                for gi in range(4):
                    col = plsc.load_gather(slab, [bufv, e_vec[gi], wv])
                    rows.at[slot][pl.ds(gi * 16, 16)] = col
                plsc.store_scatter(jb, [zeros16 + slot], zeros16 + (pk >> 9),
                                   mask=lane0)
                slot = slot + 1

                @pl.when(slot == 64)
                def _():
                    flush()

                return jnp.where(slot == 64, 0, slot)

            return lax.fori_loop(lo, hi, ebody, slot)

        lax.fori_loop(0, nl, chunk_body, jnp.int32(0))
        flush()

    return k


_sweep = _make_sweep()


def kernel(input, weight):
    wt = weight.T
    idx = input.reshape(_N).astype(jnp.int32)
    out = _sweep(wt, idx)
    return out[:_N, :_D].reshape(_BATCH, _HIST, _D)
